# baseline (speedup 1.0000x reference)
import functools

import jax
import jax.numpy as jnp
from jax import lax
from jax.experimental import pallas as pl
from jax.experimental.pallas import tpu as pltpu
from jax.experimental.pallas import tpu_sc as plsc

NUSR = 50000
NITM = 50000
NN = NUSR + NITM
NNP = 100096
DIM = 32
HALF = 16
EDG = 1600000
NLAY = 2
EPS = 0.2
TEMP = 0.2
REG_L = 1e-4
SSL_L = 0.2
BB = 4096

NC = 2
NS = 16
EK = 80
KM = 5
SUP = KM * EK
EPT = EDG // NS
NSUP = EPT // SUP
NLT = NNP // NS
WR = 136
NUP = 50048
HW = 136
CB = 128


def _spmm_body(layer, final_combine, interleaved, table, ei_h, w_h, nn_hbm,
               out_hbm, acc, src1s, dst1s, wvs, sidxs, didxs, rowss, accb, nnb,
               outb, lsems, gsems, ssems, h1b=None):
    c = lax.axis_index("c")
    s = lax.axis_index("s")
    rbase = s * NLT

    for j in range(WR):
        outb[j] = jnp.zeros((HALF,), jnp.float32)
    for t in range(NLT // WR):
        pltpu.sync_copy(outb, acc.at[pl.ds(rbase + t * WR, WR)])
    plsc.subcore_barrier()

    if interleaved:
        cofs = jnp.full((16,), 0, jnp.int32) + c
    else:
        cofs = jnp.full((16,), 0, jnp.int32) + c * NNP

    ebase = s * EPT
    last = NSUP - 1

    def fire_loads(p, g):
        off = ebase + jnp.minimum(g, last) * SUP
        pltpu.async_copy(ei_h.at[0, pl.ds(off, SUP)], src1s[p], lsems[p])
        pltpu.async_copy(ei_h.at[1, pl.ds(off, SUP)], dst1s[p], lsems[p])
        pltpu.async_copy(w_h.at[pl.ds(off, SUP)], wvs[p], lsems[p])

    def wait_loads(p):
        for buf in (src1s[p], dst1s[p], wvs[p]):
            pltpu.make_async_copy(w_h.at[pl.ds(0, SUP)], buf,
                                  lsems[p]).wait()

    def drain_scatter(p):
        for i in range(KM):
            pltpu.make_async_copy(rowss[p].at[i], acc.at[didxs[p].at[i]],
                                  ssems[p]).wait()

    def compute_idx(p):
        for i in range(KM):
            for k in range(EK // 16):
                o = i * EK + k * 16
                sv = src1s[p][pl.ds(o, 16)]
                if interleaved:
                    sidxs[p][i, pl.ds(k * 16, 16)] = sv + sv + cofs
                else:
                    sidxs[p][i, pl.ds(k * 16, 16)] = sv + cofs
                didxs[p][i, pl.ds(k * 16, 16)] = dst1s[p][pl.ds(o, 16)]

    def fire_gathers(p):
        for i in range(KM):
            pltpu.async_copy(table.at[sidxs[p].at[i]], rowss[p].at[i],
                             gsems[p])

    def wait_gathers(p):
        for i in range(KM):
            pltpu.make_async_copy(table.at[sidxs[p].at[i]], rowss[p].at[i],
                                  gsems[p]).wait()

    def multiply(p):
        for i in range(KM):
            for k in range(EK // 16):
                wvec = wvs[p][pl.ds(i * EK + k * 16, 16)]
                for l in range(16):
                    wsp = lax.gather(
                        wvec, jnp.full((16, 1), l, jnp.int32),
                        lax.GatherDimensionNumbers(
                            offset_dims=(), collapsed_slice_dims=(0,),
                            start_index_map=(0,)),
                        (1,), mode=lax.GatherScatterMode.PROMISE_IN_BOUNDS)
                    rowss[p][i, k * 16 + l] = rowss[p][i, k * 16 + l] * wsp

    def fire_scatter(p):
        for i in range(KM):
            pltpu.async_copy(rowss[p].at[i], acc.at[didxs[p].at[i]],
                             ssems[p], add=True)

    pad_row = jnp.full((16,), NNP - 1, jnp.int32)
    for p in range(2):
        for i in range(KM):
            for k in range(EK // 16):
                didxs[p][i, pl.ds(k * 16, 16)] = pad_row
        fire_scatter(p)
    fire_loads(0, 0)
    fire_loads(1, 1)
    wait_loads(0)
    drain_scatter(0)
    compute_idx(0)
    fire_gathers(0)

    def pipe_body(it, carry):
        g = 2 * it
        wait_loads(1)
        drain_scatter(1)
        compute_idx(1)
        fire_gathers(1)
        wait_gathers(0)
        multiply(0)
        fire_loads(0, g + 2)
        fire_scatter(0)
        wait_gathers(1)
        multiply(1)
        fire_loads(1, g + 3)
        fire_scatter(1)
        wait_loads(0)
        drain_scatter(0)
        compute_idx(0)
        fire_gathers(0)
        return carry

    lax.fori_loop(0, NSUP // 2, pipe_body, 0)
    wait_gathers(0)
    wait_loads(1)
    drain_scatter(1)
    plsc.subcore_barrier()

    coff = c * HALF

    def wr_body(t, carry):
        r0 = rbase + t * WR
        pltpu.sync_copy(acc.at[pl.ds(r0, WR)], accb)
        pltpu.sync_copy(nn_hbm.at[layer, pl.ds(r0, WR)], nnb)
        if final_combine:
            pltpu.sync_copy(table.at[pl.ds(c * NNP + r0, WR)], h1b)
        for j in range(WR):
            a = accb[j]
            o = a + jnp.sign(a) * nnb[j, pl.ds(coff, HALF)]
            if final_combine:
                o = (o + h1b[j]) * 0.5
            outb[j] = o
        pltpu.sync_copy(outb, out_hbm.at[pl.ds(c * NNP + r0, WR)])
        return carry

    lax.fori_loop(0, NLT // WR, wr_body, 0)


def _make_spmm(layer, final_combine, interleaved):
    mesh = plsc.VectorSubcoreMesh(core_axis_name="c", subcore_axis_name="s")
    return functools.partial(
        pl.kernel,
        functools.partial(_spmm_body, layer, final_combine, interleaved),
        out_type=jax.ShapeDtypeStruct((2 * NNP, HALF), jnp.float32),
        mesh=mesh,
        scratch_types=[
            pltpu.VMEM_SHARED((NNP, HALF), jnp.float32),
            [pltpu.VMEM((SUP,), jnp.int32) for _ in range(2)],
            [pltpu.VMEM((SUP,), jnp.int32) for _ in range(2)],
            [pltpu.VMEM((SUP,), jnp.float32) for _ in range(2)],
            [pltpu.VMEM((KM, EK), jnp.int32) for _ in range(2)],
            [pltpu.VMEM((KM, EK), jnp.int32) for _ in range(2)],
            [pltpu.VMEM((KM, EK, HALF), jnp.float32) for _ in range(2)],
            pltpu.VMEM((WR, HALF), jnp.float32),
            pltpu.VMEM((WR, DIM), jnp.float32),
            pltpu.VMEM((WR, HALF), jnp.float32),
            [pltpu.SemaphoreType.DMA for _ in range(2)],
            [pltpu.SemaphoreType.DMA for _ in range(2)],
            [pltpu.SemaphoreType.DMA for _ in range(2)],
        ] + ([pltpu.VMEM((WR, HALF), jnp.float32)] if final_combine else []),
        compiler_params=pltpu.CompilerParams(use_tc_tiling_on_sc=False),
    )()


_spmm0 = _make_spmm(0, False, True)
_spmm1 = _make_spmm(1, True, False)


def _bgather_body(fin, x0, h1, u_h, p_h, n_h, out_hbm,
                  hist, u1, p1, n1, ids, gi, rowb, oneb, zb, gsem):
    c = lax.axis_index("c")
    s = lax.axis_index("s")

    for j in range(HW):
        zb[j] = jnp.zeros((HALF,), jnp.float32)
    hrpt = NUP // NS
    for t in range(hrpt // HW):
        pltpu.sync_copy(zb, hist.at[pl.ds(s * hrpt + t * HW, HW)])
    for j in range(CB):
        oneb[j] = jnp.full((HALF,), 1.0, jnp.float32)
    plsc.subcore_barrier()

    bbase = s * (BB // NS)
    pltpu.sync_copy(u_h.at[pl.ds(bbase, 2 * CB)], u1)
    pltpu.sync_copy(p_h.at[pl.ds(bbase, 2 * CB)], p1)
    pltpu.sync_copy(n_h.at[pl.ds(bbase, 2 * CB)], n1)

    for i in range(2):
        for k in range(CB // 16):
            o = i * CB + k * 16
            sel = jnp.where(c == 0, u1[pl.ds(o, 16)], p1[pl.ds(o, 16)])
            ids[i, pl.ds(k * 16, 16)] = sel
    for i in range(2):
        pltpu.sync_copy(oneb, hist.at[ids.at[i]], add=True)
    plsc.subcore_barrier()

    sofs = jnp.full((16,), 0, jnp.int32) + c * NNP
    iofs0 = jnp.full((16,), 0, jnp.int32) + c
    iofs1 = iofs0 + 2 * NUSR

    groups = [(0, u1, fin, sofs, False), (1, p1, fin, sofs + NUSR, False),
              (2, n1, fin, sofs + NUSR, False),
              (3, u1, x0, iofs0, True), (4, p1, x0, iofs1, True),
              (5, n1, x0, iofs1, True),
              (6, u1, h1, sofs, False), (7, p1, h1, sofs + NUSR, False)]
    for g, sv, tbl, ofs, ilv in groups:
        for i in range(2):
            for k in range(CB // 16):
                o = i * CB + k * 16
                v = sv[pl.ds(o, 16)]
                if ilv:
                    gi[i, pl.ds(k * 16, 16)] = v + v + ofs
                else:
                    gi[i, pl.ds(k * 16, 16)] = v + ofs
        for i in range(2):
            pltpu.async_copy(tbl.at[gi.at[i]], rowb, gsem).wait()
            pltpu.sync_copy(
                rowb, out_hbm.at[pl.ds((c * 9 + g) * BB + bbase + i * CB, CB)])
    for i in range(2):
        pltpu.async_copy(hist.at[ids.at[i]], rowb, gsem).wait()
        pltpu.sync_copy(
            rowb, out_hbm.at[pl.ds((c * 9 + 8) * BB + bbase + i * CB, CB)])


def _make_bgather():
    mesh = plsc.VectorSubcoreMesh(core_axis_name="c", subcore_axis_name="s")
    return functools.partial(
        pl.kernel,
        _bgather_body,
        out_type=jax.ShapeDtypeStruct((2 * 9 * BB, HALF), jnp.float32),
        mesh=mesh,
        scratch_types=[
            pltpu.VMEM_SHARED((NUP, HALF), jnp.float32),
            pltpu.VMEM((2 * CB,), jnp.int32),
            pltpu.VMEM((2 * CB,), jnp.int32),
            pltpu.VMEM((2 * CB,), jnp.int32),
            pltpu.VMEM((2, CB), jnp.int32),
            pltpu.VMEM((2, CB), jnp.int32),
            pltpu.VMEM((CB, HALF), jnp.float32),
            pltpu.VMEM((CB, HALF), jnp.float32),
            pltpu.VMEM((HW, HALF), jnp.float32),
            pltpu.SemaphoreType.DMA,
        ],
        compiler_params=pltpu.CompilerParams(use_tc_tiling_on_sc=False),
    )()


_bgather = _make_bgather()


_NBN = 2000


def _noise_body(nref, oref):
    x = nref[0]
    n2 = jnp.sum(x * x, axis=1, keepdims=True)
    scale = EPS / jnp.maximum(jnp.sqrt(n2), 1e-12)
    oref[0] = x * scale


def _noise_tc(noise):
    return pl.pallas_call(
        _noise_body,
        grid=(NLAY, NN // _NBN),
        in_specs=[pl.BlockSpec((1, _NBN, DIM), lambda l, b: (l, b, 0))],
        out_specs=pl.BlockSpec((1, _NBN, DIM), lambda l, b: (l, b, 0)),
        out_shape=jax.ShapeDtypeStruct((NLAY, NNP, DIM), jnp.float32),
    )(noise)


def _bprreg_body(ga_ref, gb_ref, out_ref):
    ue = jnp.concatenate([ga_ref[0], gb_ref[0]], axis=1)
    pe = jnp.concatenate([ga_ref[1], gb_ref[1]], axis=1)
    ne = jnp.concatenate([ga_ref[2], gb_ref[2]], axis=1)
    pos = jnp.sum(ue * pe, axis=1)
    neg = jnp.sum(ue * ne, axis=1)
    x = neg - pos
    sp = jnp.maximum(x, 0.0) + jnp.log(1.0 + jnp.exp(-jnp.abs(x)))
    out_ref[0] = jnp.mean(sp)
    r = 0.0
    for i in range(3, 6):
        r = r + jnp.sum(ga_ref[i] * ga_ref[i]) + jnp.sum(gb_ref[i] * gb_ref[i])
    out_ref[1] = r


def _bprreg_tc(ga, gb):
    return pl.pallas_call(
        _bprreg_body,
        out_specs=pl.BlockSpec(memory_space=pltpu.SMEM),
        out_shape=jax.ShapeDtypeStruct((2,), jnp.float32),
    )(ga, gb)


_SBN = 512
_SCK = 512


def _ssl_body(v1a_ref, v1b_ref, v2a_ref, v2b_ref, cc_ref, cr_ref, out_ref):
    t = pl.program_id(0)
    b = pl.program_id(1)

    @pl.when(jnp.logical_and(t == 0, b == 0))
    def _():
        out_ref[0, 0] = 0.0
        out_ref[0, 1] = 0.0
        out_ref[1, 0] = 0.0
        out_ref[1, 1] = 0.0

    def _norm_rows(x):
        n = jnp.maximum(jnp.sqrt(jnp.sum(x * x, axis=1, keepdims=True)), 1e-12)
        return x / n

    v1 = _norm_rows(jnp.concatenate([v1a_ref[0], v1b_ref[0]], axis=1))

    v2blk = _norm_rows(jnp.concatenate(
        [v2a_ref[0, pl.ds(b * _SBN, _SBN)], v2b_ref[0, pl.ds(b * _SBN, _SBN)]],
        axis=1))
    pos = jnp.sum(v1 * v2blk, axis=1, keepdims=True) / TEMP

    m0 = jnp.full((_SBN, 1), -1e30, jnp.float32)
    s0 = jnp.zeros((_SBN, 1), jnp.float32)

    def cb(k, carry):
        m, sm = carry
        v2c = _norm_rows(jnp.concatenate(
            [v2a_ref[0, pl.ds(k * _SCK, _SCK)],
             v2b_ref[0, pl.ds(k * _SCK, _SCK)]], axis=1))
        logc = jnp.log(cc_ref[0, 0, pl.ds(k * _SCK, _SCK)])
        ttl = lax.dot_general(v1, v2c, (((1,), (1,)), ((), ())),
                              preferred_element_type=jnp.float32)
        ttl = ttl * (1.0 / TEMP) - logc[None, :]
        mc = jnp.max(ttl, axis=1, keepdims=True)
        mn = jnp.maximum(m, mc)
        sm = sm * jnp.exp(m - mn) + jnp.sum(jnp.exp(ttl - mn), axis=1,
                                            keepdims=True)
        return mn, sm

    m, sm = lax.fori_loop(0, BB // _SCK, cb, (m0, s0))
    lse = jnp.log(sm) + m
    cr = cr_ref[0]
    vals = (lse - pos) / cr
    out_ref[t, 0] = out_ref[t, 0] + jnp.sum(vals)
    out_ref[t, 1] = out_ref[t, 1] + jnp.sum(1.0 / cr)


def _ssl_tc(v1a, v1b, v2a, v2b, ccol, crow):
    return pl.pallas_call(
        _ssl_body,
        grid=(2, BB // _SBN),
        in_specs=[
            pl.BlockSpec((1, _SBN, HALF), lambda t, b: (t, b, 0)),
            pl.BlockSpec((1, _SBN, HALF), lambda t, b: (t, b, 0)),
            pl.BlockSpec((1, BB, HALF), lambda t, b: (t, 0, 0)),
            pl.BlockSpec((1, BB, HALF), lambda t, b: (t, 0, 0)),
            pl.BlockSpec((1, 1, BB), lambda t, b: (t, 0, 0)),
            pl.BlockSpec((1, _SBN, 1), lambda t, b: (t, b, 0)),
        ],
        out_specs=pl.BlockSpec(memory_space=pltpu.SMEM),
        out_shape=jax.ShapeDtypeStruct((2, 2), jnp.float32),
    )(v1a, v1b, v2a, v2b, ccol, crow)


def kernel(user, positive, negative, edge_index, edge_weight, noise,
           user_table, item_table):
    x0 = jnp.concatenate([user_table, item_table],
                         axis=0).reshape(2 * NN, HALF)
    ei = edge_index.astype(jnp.int32)

    nn = _noise_tc(noise)
    h1 = _spmm0(x0, ei, edge_weight, nn)
    fin = _spmm1(h1, ei, edge_weight, nn)

    g = _bgather(fin, x0, h1,
                 user.astype(jnp.int32), positive.astype(jnp.int32),
                 negative.astype(jnp.int32)).reshape(2, 9, BB, HALF)

    ga, gb = g[0], g[1]
    br = _bprreg_tc(ga[:6], gb[:6])

    cnt = jnp.stack([ga[8, :, 0], gb[8, :, 0]])
    s = _ssl_tc(ga[6:8], gb[6:8], ga[0:2], gb[0:2],
                cnt.reshape(2, 1, BB), cnt.reshape(2, BB, 1))

    bpr = br[0]
    reg = REG_L * 0.5 * br[1] / BB
    ssl = SSL_L * (s[0, 0] / s[0, 1] + s[1, 0] / s[1, 1])
    return bpr, reg, ssl


# speedup vs baseline: 1.0016x; 1.0016x over previous
import functools

import jax
import jax.numpy as jnp
from jax import lax
from jax.experimental import pallas as pl
from jax.experimental.pallas import tpu as pltpu
from jax.experimental.pallas import tpu_sc as plsc

NUSR = 50000
NITM = 50000
NN = NUSR + NITM
NNP = 100096
DIM = 32
HALF = 16
EDG = 1600000
NLAY = 2
EPS = 0.2
TEMP = 0.2
REG_L = 1e-4
SSL_L = 0.2
BB = 4096

NC = 2
NS = 16
EK = 80
KM = 5
SUP = KM * EK
EPT = EDG // NS
NSUP = EPT // SUP
NLT = NNP // NS
WR = 136
NUP = 50048
HW = 136
CB = 128


def _spmm_body(layer, final_combine, interleaved, table, ei_h, w_h, nn_hbm,
               out_hbm, acc, src1s, dst1s, wvs, sidxs, didxs, rowss, accb, nnb,
               outb, lsems, gsems, ssems, h1b=None):
    c = lax.axis_index("c")
    s = lax.axis_index("s")
    rbase = s * NLT

    for j in range(WR):
        outb[j] = jnp.zeros((HALF,), jnp.float32)
    for t in range(NLT // WR):
        pltpu.sync_copy(outb, acc.at[pl.ds(rbase + t * WR, WR)])
    plsc.subcore_barrier()

    if interleaved:
        cofs = jnp.full((16,), 0, jnp.int32) + c
    else:
        cofs = jnp.full((16,), 0, jnp.int32) + c * NNP

    ebase = s * EPT
    last = NSUP - 1

    def fire_loads(p, g):
        off = ebase + jnp.minimum(g, last) * SUP
        pltpu.async_copy(ei_h.at[0, pl.ds(off, SUP)], src1s[p], lsems[p])
        pltpu.async_copy(ei_h.at[1, pl.ds(off, SUP)], dst1s[p], lsems[p])
        pltpu.async_copy(w_h.at[pl.ds(off, SUP)], wvs[p], lsems[p])

    def wait_loads(p):
        for buf in (src1s[p], dst1s[p], wvs[p]):
            pltpu.make_async_copy(w_h.at[pl.ds(0, SUP)], buf,
                                  lsems[p]).wait()

    def drain_scatter(p):
        for i in range(KM):
            pltpu.make_async_copy(rowss[p].at[i], acc.at[didxs[p].at[i]],
                                  ssems[p]).wait()

    def compute_idx(p):
        for i in range(KM):
            for k in range(EK // 16):
                o = i * EK + k * 16
                sv = src1s[p][pl.ds(o, 16)]
                if interleaved:
                    sidxs[p][i, pl.ds(k * 16, 16)] = sv + sv + cofs
                else:
                    sidxs[p][i, pl.ds(k * 16, 16)] = sv + cofs
                didxs[p][i, pl.ds(k * 16, 16)] = dst1s[p][pl.ds(o, 16)]

    def fire_gathers(p):
        for i in range(KM):
            pltpu.async_copy(table.at[sidxs[p].at[i]], rowss[p].at[i],
                             gsems[p])

    def wait_gathers(p):
        for i in range(KM):
            pltpu.make_async_copy(table.at[sidxs[p].at[i]], rowss[p].at[i],
                                  gsems[p]).wait()

    def multiply(p):
        for i in range(KM):
            for k in range(EK // 16):
                wvec = wvs[p][pl.ds(i * EK + k * 16, 16)]
                for l in range(16):
                    wsp = jnp.zeros((16,), jnp.float32) + wvec[l]
                    rowss[p][i, k * 16 + l] = rowss[p][i, k * 16 + l] * wsp

    def fire_scatter(p):
        for i in range(KM):
            pltpu.async_copy(rowss[p].at[i], acc.at[didxs[p].at[i]],
                             ssems[p], add=True)

    pad_row = jnp.full((16,), NNP - 1, jnp.int32)
    for p in range(2):
        for i in range(KM):
            for k in range(EK // 16):
                didxs[p][i, pl.ds(k * 16, 16)] = pad_row
        fire_scatter(p)
    fire_loads(0, 0)
    fire_loads(1, 1)
    wait_loads(0)
    drain_scatter(0)
    compute_idx(0)
    fire_gathers(0)

    def pipe_body(it, carry):
        g = 2 * it
        wait_loads(1)
        drain_scatter(1)
        compute_idx(1)
        fire_gathers(1)
        wait_gathers(0)
        multiply(0)
        fire_loads(0, g + 2)
        fire_scatter(0)
        wait_gathers(1)
        multiply(1)
        fire_loads(1, g + 3)
        fire_scatter(1)
        wait_loads(0)
        drain_scatter(0)
        compute_idx(0)
        fire_gathers(0)
        return carry

    lax.fori_loop(0, NSUP // 2, pipe_body, 0)
    wait_gathers(0)
    wait_loads(1)
    drain_scatter(1)
    plsc.subcore_barrier()

    coff = c * HALF

    def wr_body(t, carry):
        r0 = rbase + t * WR
        pltpu.sync_copy(acc.at[pl.ds(r0, WR)], accb)
        pltpu.sync_copy(nn_hbm.at[layer, pl.ds(r0, WR)], nnb)
        if final_combine:
            pltpu.sync_copy(table.at[pl.ds(c * NNP + r0, WR)], h1b)
        for j in range(WR):
            a = accb[j]
            o = a + jnp.sign(a) * nnb[j, pl.ds(coff, HALF)]
            if final_combine:
                o = (o + h1b[j]) * 0.5
            outb[j] = o
        pltpu.sync_copy(outb, out_hbm.at[pl.ds(c * NNP + r0, WR)])
        return carry

    lax.fori_loop(0, NLT // WR, wr_body, 0)


def _make_spmm(layer, final_combine, interleaved):
    mesh = plsc.VectorSubcoreMesh(core_axis_name="c", subcore_axis_name="s")
    return functools.partial(
        pl.kernel,
        functools.partial(_spmm_body, layer, final_combine, interleaved),
        out_type=jax.ShapeDtypeStruct((2 * NNP, HALF), jnp.float32),
        mesh=mesh,
        scratch_types=[
            pltpu.VMEM_SHARED((NNP, HALF), jnp.float32),
            [pltpu.VMEM((SUP,), jnp.int32) for _ in range(2)],
            [pltpu.VMEM((SUP,), jnp.int32) for _ in range(2)],
            [pltpu.VMEM((SUP,), jnp.float32) for _ in range(2)],
            [pltpu.VMEM((KM, EK), jnp.int32) for _ in range(2)],
            [pltpu.VMEM((KM, EK), jnp.int32) for _ in range(2)],
            [pltpu.VMEM((KM, EK, HALF), jnp.float32) for _ in range(2)],
            pltpu.VMEM((WR, HALF), jnp.float32),
            pltpu.VMEM((WR, DIM), jnp.float32),
            pltpu.VMEM((WR, HALF), jnp.float32),
            [pltpu.SemaphoreType.DMA for _ in range(2)],
            [pltpu.SemaphoreType.DMA for _ in range(2)],
            [pltpu.SemaphoreType.DMA for _ in range(2)],
        ] + ([pltpu.VMEM((WR, HALF), jnp.float32)] if final_combine else []),
        compiler_params=pltpu.CompilerParams(use_tc_tiling_on_sc=False),
    )()


_spmm0 = _make_spmm(0, False, True)
_spmm1 = _make_spmm(1, True, False)


def _bgather_body(fin, x0, h1, u_h, p_h, n_h, out_hbm,
                  hist, u1, p1, n1, ids, gi, rowb, oneb, zb, gsem):
    c = lax.axis_index("c")
    s = lax.axis_index("s")

    for j in range(HW):
        zb[j] = jnp.zeros((HALF,), jnp.float32)
    hrpt = NUP // NS
    for t in range(hrpt // HW):
        pltpu.sync_copy(zb, hist.at[pl.ds(s * hrpt + t * HW, HW)])
    for j in range(CB):
        oneb[j] = jnp.full((HALF,), 1.0, jnp.float32)
    plsc.subcore_barrier()

    bbase = s * (BB // NS)
    pltpu.sync_copy(u_h.at[pl.ds(bbase, 2 * CB)], u1)
    pltpu.sync_copy(p_h.at[pl.ds(bbase, 2 * CB)], p1)
    pltpu.sync_copy(n_h.at[pl.ds(bbase, 2 * CB)], n1)

    for i in range(2):
        for k in range(CB // 16):
            o = i * CB + k * 16
            sel = jnp.where(c == 0, u1[pl.ds(o, 16)], p1[pl.ds(o, 16)])
            ids[i, pl.ds(k * 16, 16)] = sel
    for i in range(2):
        pltpu.sync_copy(oneb, hist.at[ids.at[i]], add=True)
    plsc.subcore_barrier()

    sofs = jnp.full((16,), 0, jnp.int32) + c * NNP
    iofs0 = jnp.full((16,), 0, jnp.int32) + c
    iofs1 = iofs0 + 2 * NUSR

    groups = [(0, u1, fin, sofs, False), (1, p1, fin, sofs + NUSR, False),
              (2, n1, fin, sofs + NUSR, False),
              (3, u1, x0, iofs0, True), (4, p1, x0, iofs1, True),
              (5, n1, x0, iofs1, True),
              (6, u1, h1, sofs, False), (7, p1, h1, sofs + NUSR, False)]
    for g, sv, tbl, ofs, ilv in groups:
        for i in range(2):
            for k in range(CB // 16):
                o = i * CB + k * 16
                v = sv[pl.ds(o, 16)]
                if ilv:
                    gi[i, pl.ds(k * 16, 16)] = v + v + ofs
                else:
                    gi[i, pl.ds(k * 16, 16)] = v + ofs
        for i in range(2):
            pltpu.async_copy(tbl.at[gi.at[i]], rowb, gsem).wait()
            pltpu.sync_copy(
                rowb, out_hbm.at[pl.ds((c * 9 + g) * BB + bbase + i * CB, CB)])
    for i in range(2):
        pltpu.async_copy(hist.at[ids.at[i]], rowb, gsem).wait()
        pltpu.sync_copy(
            rowb, out_hbm.at[pl.ds((c * 9 + 8) * BB + bbase + i * CB, CB)])


def _make_bgather():
    mesh = plsc.VectorSubcoreMesh(core_axis_name="c", subcore_axis_name="s")
    return functools.partial(
        pl.kernel,
        _bgather_body,
        out_type=jax.ShapeDtypeStruct((2 * 9 * BB, HALF), jnp.float32),
        mesh=mesh,
        scratch_types=[
            pltpu.VMEM_SHARED((NUP, HALF), jnp.float32),
            pltpu.VMEM((2 * CB,), jnp.int32),
            pltpu.VMEM((2 * CB,), jnp.int32),
            pltpu.VMEM((2 * CB,), jnp.int32),
            pltpu.VMEM((2, CB), jnp.int32),
            pltpu.VMEM((2, CB), jnp.int32),
            pltpu.VMEM((CB, HALF), jnp.float32),
            pltpu.VMEM((CB, HALF), jnp.float32),
            pltpu.VMEM((HW, HALF), jnp.float32),
            pltpu.SemaphoreType.DMA,
        ],
        compiler_params=pltpu.CompilerParams(use_tc_tiling_on_sc=False),
    )()


_bgather = _make_bgather()


_NBN = 2000


def _noise_body(nref, oref):
    x = nref[0]
    n2 = jnp.sum(x * x, axis=1, keepdims=True)
    scale = EPS / jnp.maximum(jnp.sqrt(n2), 1e-12)
    oref[0] = x * scale


def _noise_tc(noise):
    return pl.pallas_call(
        _noise_body,
        grid=(NLAY, NN // _NBN),
        in_specs=[pl.BlockSpec((1, _NBN, DIM), lambda l, b: (l, b, 0))],
        out_specs=pl.BlockSpec((1, _NBN, DIM), lambda l, b: (l, b, 0)),
        out_shape=jax.ShapeDtypeStruct((NLAY, NNP, DIM), jnp.float32),
    )(noise)


def _bprreg_body(ga_ref, gb_ref, out_ref):
    ue = jnp.concatenate([ga_ref[0], gb_ref[0]], axis=1)
    pe = jnp.concatenate([ga_ref[1], gb_ref[1]], axis=1)
    ne = jnp.concatenate([ga_ref[2], gb_ref[2]], axis=1)
    pos = jnp.sum(ue * pe, axis=1)
    neg = jnp.sum(ue * ne, axis=1)
    x = neg - pos
    sp = jnp.maximum(x, 0.0) + jnp.log(1.0 + jnp.exp(-jnp.abs(x)))
    out_ref[0] = jnp.mean(sp)
    r = 0.0
    for i in range(3, 6):
        r = r + jnp.sum(ga_ref[i] * ga_ref[i]) + jnp.sum(gb_ref[i] * gb_ref[i])
    out_ref[1] = r


def _bprreg_tc(ga, gb):
    return pl.pallas_call(
        _bprreg_body,
        out_specs=pl.BlockSpec(memory_space=pltpu.SMEM),
        out_shape=jax.ShapeDtypeStruct((2,), jnp.float32),
    )(ga, gb)


_SBN = 512
_SCK = 512


def _ssl_body(v1a_ref, v1b_ref, v2a_ref, v2b_ref, cc_ref, cr_ref, out_ref):
    t = pl.program_id(0)
    b = pl.program_id(1)

    @pl.when(jnp.logical_and(t == 0, b == 0))
    def _():
        out_ref[0, 0] = 0.0
        out_ref[0, 1] = 0.0
        out_ref[1, 0] = 0.0
        out_ref[1, 1] = 0.0

    def _norm_rows(x):
        n = jnp.maximum(jnp.sqrt(jnp.sum(x * x, axis=1, keepdims=True)), 1e-12)
        return x / n

    v1 = _norm_rows(jnp.concatenate([v1a_ref[0], v1b_ref[0]], axis=1))

    v2blk = _norm_rows(jnp.concatenate(
        [v2a_ref[0, pl.ds(b * _SBN, _SBN)], v2b_ref[0, pl.ds(b * _SBN, _SBN)]],
        axis=1))
    pos = jnp.sum(v1 * v2blk, axis=1, keepdims=True) / TEMP

    m0 = jnp.full((_SBN, 1), -1e30, jnp.float32)
    s0 = jnp.zeros((_SBN, 1), jnp.float32)

    def cb(k, carry):
        m, sm = carry
        v2c = _norm_rows(jnp.concatenate(
            [v2a_ref[0, pl.ds(k * _SCK, _SCK)],
             v2b_ref[0, pl.ds(k * _SCK, _SCK)]], axis=1))
        logc = jnp.log(cc_ref[0, 0, pl.ds(k * _SCK, _SCK)])
        ttl = lax.dot_general(v1, v2c, (((1,), (1,)), ((), ())),
                              preferred_element_type=jnp.float32)
        ttl = ttl * (1.0 / TEMP) - logc[None, :]
        mc = jnp.max(ttl, axis=1, keepdims=True)
        mn = jnp.maximum(m, mc)
        sm = sm * jnp.exp(m - mn) + jnp.sum(jnp.exp(ttl - mn), axis=1,
                                            keepdims=True)
        return mn, sm

    m, sm = lax.fori_loop(0, BB // _SCK, cb, (m0, s0))
    lse = jnp.log(sm) + m
    cr = cr_ref[0]
    vals = (lse - pos) / cr
    out_ref[t, 0] = out_ref[t, 0] + jnp.sum(vals)
    out_ref[t, 1] = out_ref[t, 1] + jnp.sum(1.0 / cr)


def _ssl_tc(v1a, v1b, v2a, v2b, ccol, crow):
    return pl.pallas_call(
        _ssl_body,
        grid=(2, BB // _SBN),
        in_specs=[
            pl.BlockSpec((1, _SBN, HALF), lambda t, b: (t, b, 0)),
            pl.BlockSpec((1, _SBN, HALF), lambda t, b: (t, b, 0)),
            pl.BlockSpec((1, BB, HALF), lambda t, b: (t, 0, 0)),
            pl.BlockSpec((1, BB, HALF), lambda t, b: (t, 0, 0)),
            pl.BlockSpec((1, 1, BB), lambda t, b: (t, 0, 0)),
            pl.BlockSpec((1, _SBN, 1), lambda t, b: (t, b, 0)),
        ],
        out_specs=pl.BlockSpec(memory_space=pltpu.SMEM),
        out_shape=jax.ShapeDtypeStruct((2, 2), jnp.float32),
    )(v1a, v1b, v2a, v2b, ccol, crow)


def kernel(user, positive, negative, edge_index, edge_weight, noise,
           user_table, item_table):
    x0 = jnp.concatenate([user_table, item_table],
                         axis=0).reshape(2 * NN, HALF)
    ei = edge_index.astype(jnp.int32)

    nn = _noise_tc(noise)
    h1 = _spmm0(x0, ei, edge_weight, nn)
    fin = _spmm1(h1, ei, edge_weight, nn)

    g = _bgather(fin, x0, h1,
                 user.astype(jnp.int32), positive.astype(jnp.int32),
                 negative.astype(jnp.int32)).reshape(2, 9, BB, HALF)

    ga, gb = g[0], g[1]
    br = _bprreg_tc(ga[:6], gb[:6])

    cnt = jnp.stack([ga[8, :, 0], gb[8, :, 0]])
    s = _ssl_tc(ga[6:8], gb[6:8], ga[0:2], gb[0:2],
                cnt.reshape(2, 1, BB), cnt.reshape(2, BB, 1))

    bpr = br[0]
    reg = REG_L * 0.5 * br[1] / BB
    ssl = SSL_L * (s[0, 0] / s[0, 1] + s[1, 0] / s[1, 1])
    return bpr, reg, ssl
